# baseline (speedup 1.0000x reference)
import functools

import jax
import jax.numpy as jnp
from jax import lax
from jax.experimental import pallas as pl
from jax.experimental.pallas import tpu as pltpu
from jax.experimental.pallas import tpu_sc as plsc

BATCH = 16384
FEAT = 32
NUM_CLASSES = 1000000
NC, NS, L = 2, 16, 16
NW = NC * NS
BPW = BATCH // NW
BPC = BATCH // NC
NCOL = (NUM_CLASSES + 127) // 128
KPAD = 512
ECAP = 768
RING = 16
DEPTH = 12
LAST_BASE = NUM_CLASSES - 128


def _sc_partials(features_t, labels, centers_t):
    mesh = plsc.VectorSubcoreMesh(core_axis_name="c", subcore_axis_name="s")

    @functools.partial(
        pl.kernel,
        mesh=mesh,
        out_type=(jax.ShapeDtypeStruct((NW * L,), jnp.float32),
                  jax.ShapeDtypeStruct((BATCH * FEAT,), jnp.float32)),
        scratch_types=[
            pltpu.VMEM((BPC,), jnp.int32),
            pltpu.VMEM((KPAD,), jnp.int32),
            pltpu.VMEM((KPAD,), jnp.int32),
            pltpu.VMEM((ECAP,), jnp.int32),
            pltpu.VMEM((ECAP,), jnp.int32),
            pltpu.VMEM((ECAP,), jnp.int32),
            pltpu.VMEM((RING, FEAT, 128), jnp.float32),
            pltpu.VMEM((FEAT,), jnp.float32),
            pltpu.VMEM((BPW * FEAT,), jnp.float32),
            pltpu.VMEM((FEAT, BPW), jnp.float32),
            pltpu.VMEM((L,), jnp.float32),
            pltpu.SemaphoreType.DMA,
            pltpu.SemaphoreType.DMA,
        ],
        compiler_params=pltpu.CompilerParams(
            use_tc_tiling_on_sc=True, needs_layout_passes=False),
    )
    def k(feat_hbm, lab_hbm, cent_hbm, out_hbm, stage_hbm, lab_v, flag_v,
          klist_v, ecol_v, elane_v, eidx_v, ring_v, tmp_v, gath_v, feat_v,
          acc_v, fsem, gsem):
        sc = lax.axis_index("c")
        t = lax.axis_index("s")
        wid = t * NC + sc
        lane_iota = jax.lax.broadcasted_iota(jnp.int32, (L,), 0)
        fidx = lane_iota

        pltpu.sync_copy(lab_hbm.at[pl.ds(pl.multiple_of(sc * BPC, 128), BPC)],
                        lab_v)
        zeros = jnp.zeros((L,), jnp.int32)
        for v in range(KPAD // L):
            flag_v[pl.ds(v * L, L)] = zeros
        ones = jnp.ones((L,), jnp.int32)

        def scan_body(v, cnt):
            labs = lab_v[pl.ds(v * L, L)]
            col = labs >> 7
            cb = jnp.minimum((labs >> 7) << 7, LAST_BASE)
            mine = (col & (NS - 1)) == t
            plsc.store_scatter(flag_v, [col >> 4], ones, mask=mine)
            plsc.store_compressed(ecol_v.at[pl.ds(cnt, L)], col, mask=mine)
            plsc.store_compressed(elane_v.at[pl.ds(cnt, L)], labs - cb,
                                  mask=mine)
            plsc.store_compressed(eidx_v.at[pl.ds(cnt, L)],
                                  lane_iota + v * L, mask=mine)
            nm = plsc.all_reduce_population_count(mine)
            return cnt + nm[0]

        cnt = lax.fori_loop(0, BPC // L, scan_body, jnp.int32(0))

        def compact_body(v, ccnt):
            f = flag_v[pl.ds(v * L, L)]
            m = f > 0
            plsc.store_compressed(klist_v.at[pl.ds(ccnt, L)],
                                  lane_iota + v * L, mask=m)
            nm = plsc.all_reduce_population_count(m)
            return ccnt + nm[0]

        ccnt = lax.fori_loop(0, KPAD // L, compact_body, jnp.int32(0))

        def kth_col(m):
            kk = plsc.load_gather(klist_v, [jnp.broadcast_to(m, (L,))])
            return (kk[0] << 4) | t

        def fire(col, slot):
            cb = pl.multiple_of(jnp.minimum(col << 7, LAST_BASE), 128)
            pltpu.make_async_copy(
                cent_hbm.at[:, pl.ds(cb, 128)], ring_v.at[slot], gsem,
            ).start()

        def drain_one(slot):
            pltpu.make_async_copy(
                cent_hbm.at[:, pl.ds(0, 128)], ring_v.at[slot], gsem,
            ).wait()

        def prime(m, _):
            @pl.when(m < ccnt)
            def _():
                fire(kth_col(m), lax.rem(m, RING))
            return 0

        lax.fori_loop(0, DEPTH, prime, 0)

        nvec = (cnt + L - 1) >> 4

        def col_body(m, _):
            slot = lax.rem(m, RING)
            drain_one(slot)
            col = kth_col(m)

            def scan_entries(v, _):
                e0 = v * L
                ecols = ecol_v[pl.ds(e0, L)]
                match = (ecols == col) & ((lane_iota + e0) < cnt)

                def extract(carry):
                    msk, c = carry
                    ln = plsc.all_reduce_ffs(msk)
                    e = jnp.broadcast_to(e0, (L,)) + ln
                    l = plsc.load_gather(elane_v, [e])
                    i = plsc.load_gather(eidx_v, [e])
                    c0 = plsc.load_gather(ring_v.at[slot], [fidx, l])
                    c1 = plsc.load_gather(ring_v.at[slot], [fidx + L, l])
                    tmp_v[pl.ds(0, L)] = c0
                    tmp_v[pl.ds(L, L)] = c1
                    gi = (sc * BPC + i[0]) * FEAT
                    pltpu.sync_copy(tmp_v, stage_hbm.at[pl.ds(gi, FEAT)])
                    msk = msk & (lane_iota != ln)
                    return msk, c

                def has_more(carry):
                    msk, _c = carry
                    return plsc.all_reduce_population_count(msk)[0] > 0

                lax.while_loop(has_more, extract, (match, jnp.int32(0)))
                return 0

            lax.fori_loop(0, nvec, scan_entries, 0)

            @pl.when(m + DEPTH < ccnt)
            def _():
                fire(kth_col(m + DEPTH), lax.rem(m + DEPTH, RING))

            return 0

        lax.fori_loop(0, ccnt, col_body, 0)

        plsc.subcore_barrier()
        base = pl.multiple_of(sc * BPC + t * BPW, 128)
        fcp = pltpu.make_async_copy(
            feat_hbm.at[:, pl.ds(base, BPW)], feat_v, fsem)
        fcp.start()
        pltpu.sync_copy(stage_hbm.at[pl.ds(base * FEAT, BPW * FEAT)], gath_v)
        fcp.wait()

        def body(j, acc):
            c0 = gath_v[pl.ds(j * FEAT, L)]
            c1 = gath_v[pl.ds(j * FEAT + L, L)]
            col = jnp.broadcast_to(j, (L,))
            f0 = plsc.load_gather(feat_v, [fidx, col])
            f1 = plsc.load_gather(feat_v, [fidx + L, col])
            d0 = f0 - c0
            d1 = f1 - c1
            return acc + d0 * d0 + d1 * d1

        acc = lax.fori_loop(0, BPW, body, jnp.zeros((L,), jnp.float32))
        acc_v[...] = acc
        pltpu.sync_copy(acc_v, out_hbm.at[pl.ds(wid * L, L)])

    return k(features_t, labels, centers_t)[0]


def _tc_reduce(partials):
    def body(p_ref, o_ref):
        o_ref[0, 0] = jnp.sum(p_ref[...]) * (1.0 / BATCH)

    out = pl.pallas_call(
        body,
        out_shape=jax.ShapeDtypeStruct((1, 1), jnp.float32),
        out_specs=pl.BlockSpec(memory_space=pltpu.SMEM),
    )(partials)
    return out.reshape(())


def kernel(features, labels, centers):
    labels = labels.astype(jnp.int32)
    partials = _sc_partials(features.T, labels, centers.T)
    return _tc_reduce(partials)


# speedup vs baseline: 3.2465x; 3.2465x over previous
import functools

import jax
import jax.numpy as jnp
from jax import lax
from jax.experimental import pallas as pl
from jax.experimental.pallas import tpu as pltpu
from jax.experimental.pallas import tpu_sc as plsc

BATCH = 16384
FEAT = 32
NUM_CLASSES = 1000000
NC, NS, L = 2, 16, 16
NW = NC * NS
BPW = BATCH // NW
RING = 16
DEPTH = 12
LAST_BASE = NUM_CLASSES - 128


def _sc_partials(features_t, labels, centers_t):
    mesh = plsc.VectorSubcoreMesh(core_axis_name="c", subcore_axis_name="s")

    @functools.partial(
        pl.kernel,
        mesh=mesh,
        out_type=jax.ShapeDtypeStruct((NW * L,), jnp.float32),
        scratch_types=[
            pltpu.VMEM((BPW,), jnp.int32),
            pltpu.VMEM((FEAT, BPW), jnp.float32),
            pltpu.VMEM((RING, FEAT, 128), jnp.float32),
            pltpu.VMEM((L,), jnp.float32),
            pltpu.SemaphoreType.DMA,
            pltpu.SemaphoreType.DMA,
        ],
        compiler_params=pltpu.CompilerParams(
            use_tc_tiling_on_sc=True, needs_layout_passes=False),
    )
    def k(feat_hbm, lab_hbm, cent_hbm, out_hbm, idx_v, feat_v, ring_v, acc_v,
          fsem, gsem):
        wid = lax.axis_index("s") * NC + lax.axis_index("c")
        base = pl.multiple_of(wid * BPW, 128)
        pltpu.sync_copy(lab_hbm.at[pl.ds(base, BPW)], idx_v)
        fcp = pltpu.make_async_copy(
            feat_hbm.at[:, pl.ds(base, BPW)], feat_v, fsem)
        fcp.start()

        fidx = jax.lax.broadcasted_iota(jnp.int32, (L,), 0)

        def col_base(r):
            return jnp.minimum((r >> 7) << 7, LAST_BASE)

        def fire(r, slot):
            pltpu.make_async_copy(
                cent_hbm.at[:, pl.ds(pl.multiple_of(col_base(r), 128), 128)],
                ring_v.at[slot],
                gsem,
            ).start()

        def drain_one(slot):
            pltpu.make_async_copy(
                cent_hbm.at[:, pl.ds(0, 128)],
                ring_v.at[slot],
                gsem,
            ).wait()

        lab0 = idx_v[pl.ds(0, L)]
        for j in range(DEPTH):
            fire(lab0[j], j)
        fcp.wait()

        def body(g, acc):
            j0 = g * L
            lab_vec = idx_v[pl.ds(j0, L)]
            nxt_off = jnp.minimum(j0 + L, BPW - L)
            lab_nxt = idx_v[pl.ds(nxt_off, L)]
            for jj in range(L):
                j = j0 + jj
                slot = j % RING
                r = lab_vec[jj]
                lane = jnp.broadcast_to(r - col_base(r), (L,))
                col = jnp.broadcast_to(j, (L,))
                f0 = plsc.load_gather(feat_v, [fidx, col])
                f1 = plsc.load_gather(feat_v, [fidx + L, col])
                drain_one(slot)
                c0 = plsc.load_gather(ring_v.at[slot], [fidx, lane])
                c1 = plsc.load_gather(ring_v.at[slot], [fidx + L, lane])
                d0 = f0 - c0
                d1 = f1 - c1
                acc = acc + d0 * d0 + d1 * d1
                if jj + DEPTH < L:
                    r_nxt = lab_vec[jj + DEPTH]
                else:
                    r_nxt = lab_nxt[jj + DEPTH - L]
                nxt = j + DEPTH

                @pl.when(nxt < BPW)
                def _():
                    fire(r_nxt, nxt % RING)

            return acc

        acc = lax.fori_loop(0, BPW // L, body, jnp.zeros((L,), jnp.float32))
        acc_v[...] = acc
        pltpu.sync_copy(acc_v, out_hbm.at[pl.ds(wid * L, L)])

    return k(features_t, labels, centers_t)


def _tc_reduce(partials):
    def body(p_ref, o_ref):
        o_ref[0, 0] = jnp.sum(p_ref[...]) * (1.0 / BATCH)

    out = pl.pallas_call(
        body,
        out_shape=jax.ShapeDtypeStruct((1, 1), jnp.float32),
        out_specs=pl.BlockSpec(memory_space=pltpu.SMEM),
    )(partials)
    return out.reshape(())


def kernel(features, labels, centers):
    labels = labels.astype(jnp.int32)
    partials = _sc_partials(features.T, labels, centers.T)
    return _tc_reduce(partials)
